# baseline (speedup 1.0000x reference)
import functools

import jax
import jax.numpy as jnp
from jax import lax
from jax.experimental import pallas as pl
from jax.experimental.pallas import tpu as pltpu
from jax.experimental.pallas import tpu_sc as plsc

_BATCH = 16384
_NC = 2
_NS = 16
_NW = _NC * _NS
_BPW = _BATCH // _NW
_CHUNK = 128
_NCHUNK = _BPW // _CHUNK
_L = 16
_NVREG = _BPW // _L


def _log1p_exp_neg_abs(x):
    u = jnp.exp(-jnp.abs(x))
    s = u / (2.0 + u)
    s2 = s * s
    poly = 1.0 + s2 * (
        jnp.float32(1 / 3) + s2 * (
            jnp.float32(1 / 5) + s2 * (
                jnp.float32(1 / 7) + s2 * jnp.float32(1 / 9))))
    return (2.0 * s) * poly


def _softplus(x):
    return jnp.maximum(x, 0.0) + _log1p_exp_neg_abs(x)


def _lane_sum(v):
    lanes = lax.iota(jnp.int32, _L)
    for k in (8, 4, 2, 1):
        v = v + jnp.take(v, lanes ^ k)
    return v


def _sc_irt(student_ids, question_ids, labels, ability, difficulty):
    mesh = plsc.VectorSubcoreMesh(core_axis_name="c", subcore_axis_name="s")

    @functools.partial(
        pl.kernel,
        mesh=mesh,
        out_type=(
            jax.ShapeDtypeStruct((_BATCH,), jnp.float32),
            jax.ShapeDtypeStruct((_NW, _L), jnp.float32),
        ),
        scratch_types=[
            pltpu.VMEM((_BPW,), jnp.int32),
            pltpu.VMEM((_BPW,), jnp.int32),
            pltpu.VMEM((_BPW,), jnp.float32),
            pltpu.VMEM((_BPW,), jnp.float32),
            pltpu.VMEM((_BPW,), jnp.float32),
            pltpu.VMEM((_BPW,), jnp.float32),
            pltpu.VMEM((_L,), jnp.float32),
            pltpu.SemaphoreType.DMA,
            pltpu.SemaphoreType.DMA,
        ],
    )
    def irt_kernel(sid_hbm, qid_hbm, lbl_hbm, ab_hbm, df_hbm,
                   pred_out, loss_out,
                   sidx_v, qidx_v, a_v, d_v, l_v, p_v, part_v,
                   sem_a, sem_d):
        cid = lax.axis_index("c")
        sid = lax.axis_index("s")
        wid = sid * _NC + cid
        base = wid * _BPW

        pltpu.sync_copy(sid_hbm.at[pl.ds(base, _BPW)], sidx_v)
        pltpu.sync_copy(qid_hbm.at[pl.ds(base, _BPW)], qidx_v)
        pltpu.sync_copy(lbl_hbm.at[pl.ds(base, _BPW)], l_v)

        copies = []
        for j in range(_NCHUNK):
            sl = pl.ds(j * _CHUNK, _CHUNK)
            copies.append(
                pltpu.async_copy(ab_hbm.at[sidx_v.at[sl]], a_v.at[sl], sem_a))
            copies.append(
                pltpu.async_copy(df_hbm.at[qidx_v.at[sl]], d_v.at[sl], sem_d))
        for c in copies:
            c.wait()

        inv_batch = jnp.float32(1.0 / _BATCH)

        def body(i, acc):
            sl = pl.ds(pl.multiple_of(i * _L, _L), _L)
            sa = _softplus(a_v[sl])
            sd = _softplus(d_v[sl])
            p = sa - sd
            p_v[sl] = p
            t = (jnp.maximum(p, 0.0) - p * l_v[sl]
                 + _log1p_exp_neg_abs(p))
            return acc + t * inv_batch

        acc = lax.fori_loop(0, _NVREG, body, jnp.zeros((_L,), jnp.float32))

        pltpu.sync_copy(p_v, pred_out.at[pl.ds(base, _BPW)])

        part_v[...] = _lane_sum(acc)
        pltpu.sync_copy(part_v, loss_out.at[wid])

    return irt_kernel(student_ids, question_ids, labels, ability, difficulty)


def kernel(student_ids, question_ids_collapsed, labels, ability, difficulty):
    predictions, loss_parts = _sc_irt(
        student_ids, question_ids_collapsed, labels, ability, difficulty)
    avg_loss = jnp.sum(loss_parts[:, 0])
    return (avg_loss, predictions)


# speedup vs baseline: 1.0994x; 1.0994x over previous
import functools

import jax
import jax.numpy as jnp
from jax import lax
from jax.experimental import pallas as pl
from jax.experimental.pallas import tpu as pltpu
from jax.experimental.pallas import tpu_sc as plsc

_BATCH = 16384
_NC = 2
_NS = 16
_NW = _NC * _NS
_BPW = _BATCH // _NW


def _sc_gather(student_ids, question_ids, ability, difficulty):
    mesh = plsc.VectorSubcoreMesh(core_axis_name="c", subcore_axis_name="s")

    @functools.partial(
        pl.kernel,
        mesh=mesh,
        out_type=(
            jax.ShapeDtypeStruct((_BATCH,), jnp.float32),
            jax.ShapeDtypeStruct((_BATCH,), jnp.float32),
        ),
        scratch_types=[
            pltpu.VMEM((_BPW,), jnp.int32),
            pltpu.VMEM((_BPW,), jnp.int32),
            pltpu.VMEM((_BPW,), jnp.float32),
            pltpu.VMEM((_BPW,), jnp.float32),
            pltpu.SemaphoreType.DMA,
            pltpu.SemaphoreType.DMA,
            pltpu.SemaphoreType.DMA,
            pltpu.SemaphoreType.DMA,
        ],
    )
    def gather_kernel(sid_hbm, qid_hbm, ab_hbm, df_hbm, a_out, d_out,
                      sidx_v, qidx_v, a_v, d_v,
                      sem_si, sem_qi, sem_a, sem_d):
        wid = lax.axis_index("s") * _NC + lax.axis_index("c")
        base = wid * _BPW
        sl = pl.ds(base, _BPW)
        c_si = pltpu.async_copy(sid_hbm.at[sl], sidx_v, sem_si)
        c_qi = pltpu.async_copy(qid_hbm.at[sl], qidx_v, sem_qi)
        c_si.wait()
        c_a = pltpu.async_copy(ab_hbm.at[sidx_v], a_v, sem_a)
        c_qi.wait()
        c_d = pltpu.async_copy(df_hbm.at[qidx_v], d_v, sem_d)
        c_a.wait()
        c_ao = pltpu.async_copy(a_v, a_out.at[sl], sem_a)
        c_d.wait()
        c_do = pltpu.async_copy(d_v, d_out.at[sl], sem_d)
        c_ao.wait()
        c_do.wait()

    return gather_kernel(student_ids, question_ids, ability, difficulty)


def _tc_finish(a_gathered, d_gathered, labels):
    rows = 128
    cols = _BATCH // rows

    def body(a_ref, d_ref, l_ref, pred_ref, loss_ref):
        sa = jax.nn.softplus(a_ref[...])
        sd = jax.nn.softplus(d_ref[...])
        p = sa - sd
        pred_ref[...] = p
        t = (jnp.maximum(p, 0.0) - p * l_ref[...]
             + jnp.log1p(jnp.exp(-jnp.abs(p))))
        loss_ref[...] = jnp.sum(t).reshape(1, 1) * (1.0 / _BATCH)

    pred, loss = pl.pallas_call(
        body,
        out_shape=(
            jax.ShapeDtypeStruct((rows, cols), jnp.float32),
            jax.ShapeDtypeStruct((1, 1), jnp.float32),
        ),
    )(a_gathered.reshape(rows, cols),
      d_gathered.reshape(rows, cols),
      labels.reshape(rows, cols))
    return loss[0, 0], pred.reshape(_BATCH)


def kernel(student_ids, question_ids_collapsed, labels, ability, difficulty):
    a_vals, d_vals = _sc_gather(student_ids, question_ids_collapsed,
                                ability, difficulty)
    avg_loss, predictions = _tc_finish(a_vals, d_vals, labels)
    return (avg_loss, predictions)
